# baseline (speedup 1.0000x reference)
import functools

import jax
import jax.numpy as jnp
from jax import lax
from jax.experimental import pallas as pl
from jax.experimental.pallas import tpu as pltpu
from jax.experimental.pallas import tpu_sc as plsc


def _make_lookup(B, V, D):
  info = plsc.get_sparse_core_info()
  NC, NS = info.num_cores, info.num_subcores
  N0 = 504
  N1 = B // NS - N0
  split = N0 * NS

  mesh = plsc.VectorSubcoreMesh(core_axis_name="c", subcore_axis_name="s")

  ROWS_PER_TILE = 64
  full_tiles = V // ROWS_PER_TILE
  tail_rows = V - full_tiles * ROWS_PER_TILE

  @functools.partial(
      pl.kernel,
      mesh=mesh,
      out_type=jax.ShapeDtypeStruct((B, D), jnp.float32),
      scratch_types=[
          pltpu.VMEM((max(N0, N1),), jnp.int32),
          pltpu.VMEM((max(N0, N1), D), jnp.float32),
          pltpu.VMEM_SHARED((V, D), jnp.float32),
          pltpu.SemaphoreType.DMA,
          pltpu.SemaphoreType.DMA,
      ],
  )
  def lookup(t_hbm, table_hbm, out_hbm, idx_v, rows_v, table_sp,
             g0s, isem):
    c = lax.axis_index("c")
    s = lax.axis_index("s")

    def run(base, n):
      ia = pltpu.async_copy(
          t_hbm.at[pl.ds(base, n)], idx_v.at[pl.ds(0, n)], isem)

      @pl.when(s < full_tiles)
      def _():
        r = s * ROWS_PER_TILE
        pltpu.sync_copy(table_hbm.at[pl.ds(r, ROWS_PER_TILE)],
                        table_sp.at[pl.ds(r, ROWS_PER_TILE)])

      if tail_rows:
        @pl.when(s == full_tiles)
        def _():
          r = full_tiles * ROWS_PER_TILE
          pltpu.sync_copy(table_hbm.at[pl.ds(r, tail_rows)],
                          table_sp.at[pl.ds(r, tail_rows)])

      plsc.subcore_barrier()
      ia.wait()
      pltpu.async_copy(
          table_sp.at[idx_v.at[pl.ds(0, n)]],
          rows_v.at[pl.ds(0, n)], g0s).wait()
      pltpu.sync_copy(rows_v.at[pl.ds(0, n)], out_hbm.at[pl.ds(base, n)])

    @pl.when(c == 0)
    def _():
      run(s * N0, N0)

    @pl.when(c == 1)
    def _():
      run(split + s * N1, N1)

  return lookup


def kernel(t, table):
  B, = t.shape
  V, D = table.shape
  lookup = _make_lookup(B, V, D)
  return lookup(t.astype(jnp.int32), table)


# speedup vs baseline: 1.0095x; 1.0095x over previous
import functools

import jax
import jax.numpy as jnp
from jax import lax
from jax.experimental import pallas as pl
from jax.experimental.pallas import tpu as pltpu
from jax.experimental.pallas import tpu_sc as plsc


def _make_lookup(B, V, D):
  info = plsc.get_sparse_core_info()
  NC, NS = info.num_cores, info.num_subcores
  N0 = 496
  N1 = B // NS - N0
  split = N0 * NS

  mesh = plsc.VectorSubcoreMesh(core_axis_name="c", subcore_axis_name="s")

  ROWS_PER_TILE = 64
  full_tiles = V // ROWS_PER_TILE
  tail_rows = V - full_tiles * ROWS_PER_TILE

  @functools.partial(
      pl.kernel,
      mesh=mesh,
      out_type=jax.ShapeDtypeStruct((B, D), jnp.float32),
      scratch_types=[
          pltpu.VMEM((max(N0, N1),), jnp.int32),
          pltpu.VMEM((max(N0, N1), D), jnp.float32),
          pltpu.VMEM_SHARED((V, D), jnp.float32),
          pltpu.SemaphoreType.DMA,
          pltpu.SemaphoreType.DMA,
      ],
  )
  def lookup(t_hbm, table_hbm, out_hbm, idx_v, rows_v, table_sp,
             g0s, isem):
    c = lax.axis_index("c")
    s = lax.axis_index("s")
    nmax = max(N0, N1)
    base = jnp.where(c == 0, s * N0, split + s * N1)

    ia = pltpu.async_copy(
        t_hbm.at[pl.ds(base, nmax)], idx_v.at[pl.ds(0, nmax)], isem)

    @pl.when(s < full_tiles)
    def _():
      r = s * ROWS_PER_TILE
      pltpu.sync_copy(table_hbm.at[pl.ds(r, ROWS_PER_TILE)],
                      table_sp.at[pl.ds(r, ROWS_PER_TILE)])

    if tail_rows:
      @pl.when(s == full_tiles)
      def _():
        r = full_tiles * ROWS_PER_TILE
        pltpu.sync_copy(table_hbm.at[pl.ds(r, tail_rows)],
                        table_sp.at[pl.ds(r, tail_rows)])

    plsc.subcore_barrier()
    ia.wait()

    def run(n):
      pltpu.async_copy(
          table_sp.at[idx_v.at[pl.ds(0, n)]],
          rows_v.at[pl.ds(0, n)], g0s).wait()
      pltpu.sync_copy(rows_v.at[pl.ds(0, n)], out_hbm.at[pl.ds(base, n)])

    @pl.when(c == 0)
    def _():
      run(N0)

    @pl.when(c == 1)
    def _():
      run(N1)

  return lookup


def kernel(t, table):
  B, = t.shape
  V, D = table.shape
  lookup = _make_lookup(B, V, D)
  return lookup(t.astype(jnp.int32), table)
